# baseline (speedup 1.0000x reference)
import functools

import jax
import jax.numpy as jnp
from jax import lax
from jax.experimental import pallas as pl
from jax.experimental.pallas import tpu as pltpu
from jax.experimental.pallas import tpu_sc as plsc

D_MODEL = 128
SCALE = float(D_MODEL) ** 0.5

NC = 2
NS = 16
NW = NC * NS
G = 128
NB = 2


@functools.lru_cache(maxsize=None)
def _emb_kernel(B: int):
    n_per_w = B // NW
    n_groups = n_per_w // G
    assert n_groups % NB == 0

    mesh = plsc.VectorSubcoreMesh(core_axis_name="c", subcore_axis_name="s")

    @functools.partial(
        pl.kernel,
        mesh=mesh,
        out_type=jax.ShapeDtypeStruct((B, D_MODEL), jnp.float32),
        scratch_types=[
            pltpu.VMEM((n_groups, G), jnp.int32),
            *[pltpu.VMEM((G, D_MODEL), jnp.float32) for _ in range(NB)],
            *[pltpu.SemaphoreType.DMA for _ in range(NB)],
        ],
    )
    def k(x_hbm, table_hbm, out_hbm, idx_v, *bufs_and_sems):
        rows = bufs_and_sems[0:NB]
        osem = bufs_and_sems[NB:2 * NB]

        wid = lax.axis_index("s") * NC + lax.axis_index("c")
        base = wid * n_per_w
        pltpu.sync_copy(x_hbm.at[wid], idx_v)

        def step(i, carry):
            for b in range(NB):
                gg = i * NB + b

                @pl.when(gg >= NB)
                def _():
                    pltpu.make_async_copy(
                        rows[b],
                        out_hbm.at[pl.ds(base + (gg - NB) * G, G)],
                        osem[b]).wait()

                pltpu.async_copy(table_hbm.at[idx_v.at[gg]], rows[b],
                                 osem[b]).wait()

                def row(r, c):
                    for j in range(D_MODEL // 16):
                        sl = pl.ds(j * 16, 16)
                        rows[b][r, sl] = rows[b][r, sl] * SCALE
                    return c

                lax.fori_loop(0, G, row, 0, unroll=4)

                pltpu.async_copy(rows[b],
                                 out_hbm.at[pl.ds(base + gg * G, G)],
                                 osem[b])
            return carry

        lax.fori_loop(0, n_groups // NB, step, 0)

        for b in range(NB):
            gg = n_groups - NB + b
            pltpu.make_async_copy(rows[b],
                                  out_hbm.at[pl.ds(base + gg * G, G)],
                                  osem[b]).wait()

    return k


def kernel(x, table):
    s0, s1 = x.shape
    B = s0 * s1
    xi = x.reshape(NW, B // (NW * G), G).astype(jnp.int32)
    out = _emb_kernel(B)(xi, table)
    return out.reshape(s0, s1, D_MODEL)


# speedup vs baseline: 1.3276x; 1.3276x over previous
import functools

import jax
import jax.numpy as jnp
from jax import lax
from jax.experimental import pallas as pl
from jax.experimental.pallas import tpu as pltpu
from jax.experimental.pallas import tpu_sc as plsc

D_MODEL = 128
SCALE = float(D_MODEL) ** 0.5

NC = 2
NS = 16
NW = NC * NS
G = 128


@functools.lru_cache(maxsize=None)
def _emb_kernel(B: int):
    n_per_w = B // NW
    n_groups = n_per_w // G
    assert n_groups % 2 == 0

    mesh = plsc.VectorSubcoreMesh(core_axis_name="c", subcore_axis_name="s")

    @functools.partial(
        pl.kernel,
        mesh=mesh,
        out_type=jax.ShapeDtypeStruct((B, D_MODEL), jnp.float32),
        scratch_types=[
            pltpu.VMEM((n_groups, G), jnp.int32),
            pltpu.VMEM((G, D_MODEL), jnp.float32),
            pltpu.VMEM((G, D_MODEL), jnp.float32),
            pltpu.SemaphoreType.DMA,
            pltpu.SemaphoreType.DMA,
            pltpu.SemaphoreType.DMA,
            pltpu.SemaphoreType.DMA,
        ],
    )
    def k(x_hbm, table_hbm, out_hbm, idx_v, r0, r1, g0, g1, o0, o1):
        rows = (r0, r1)
        gsem = (g0, g1)
        osem = (o0, o1)

        wid = lax.axis_index("s") * NC + lax.axis_index("c")
        base = wid * n_per_w
        pltpu.sync_copy(x_hbm.at[wid], idx_v)

        pltpu.async_copy(table_hbm.at[idx_v.at[0]], rows[0], gsem[0])

        def step(i, carry):
            for b in range(2):
                gg = i * 2 + b
                bn = 1 - b

                pltpu.make_async_copy(table_hbm.at[pl.ds(0, G)],
                                      rows[b], gsem[b]).wait()

                @pl.when(gg + 1 < n_groups)
                def _():
                    @pl.when(gg >= 1)
                    def _():
                        pltpu.make_async_copy(
                            rows[bn],
                            out_hbm.at[pl.ds(base + (gg - 1) * G, G)],
                            osem[bn]).wait()

                    pltpu.async_copy(table_hbm.at[idx_v.at[gg + 1]],
                                     rows[bn], gsem[bn])

                def row(r, c):
                    for j in range(D_MODEL // 16):
                        sl = pl.ds(j * 16, 16)
                        rows[b][r, sl] = rows[b][r, sl] * SCALE
                    return c

                lax.fori_loop(0, G, row, 0)

                pltpu.async_copy(rows[b],
                                 out_hbm.at[pl.ds(base + gg * G, G)],
                                 osem[b])
            return carry

        lax.fori_loop(0, n_groups // 2, step, 0)

        for b in range(2):
            gg = n_groups - 2 + b
            pltpu.make_async_copy(rows[b],
                                  out_hbm.at[pl.ds(base + gg * G, G)],
                                  osem[b]).wait()

    return k


def kernel(x, table):
    s0, s1 = x.shape
    B = s0 * s1
    xi = x.reshape(NW, B // (NW * G), G).astype(jnp.int32)
    out = _emb_kernel(B)(xi, table)
    return out.reshape(s0, s1, D_MODEL)


# speedup vs baseline: 1.5095x; 1.1370x over previous
import functools

import jax
import jax.numpy as jnp
from jax import lax
from jax.experimental import pallas as pl
from jax.experimental.pallas import tpu as pltpu
from jax.experimental.pallas import tpu_sc as plsc

D_MODEL = 128
SCALE = float(D_MODEL) ** 0.5

NC = 2
NS = 16
NW = NC * NS
G = 128
NBUF = 5
PF = 2


@functools.lru_cache(maxsize=None)
def _emb_kernel(B: int):
    n_per_w = B // NW
    n_groups = n_per_w // G
    assert n_groups % NBUF == 0

    mesh = plsc.VectorSubcoreMesh(core_axis_name="c", subcore_axis_name="s")

    @functools.partial(
        pl.kernel,
        mesh=mesh,
        out_type=jax.ShapeDtypeStruct((B, D_MODEL), jnp.float32),
        scratch_types=[
            pltpu.VMEM((n_groups, G), jnp.int32),
            *[pltpu.VMEM((G, D_MODEL), jnp.float32) for _ in range(NBUF)],
            *[pltpu.SemaphoreType.DMA for _ in range(2 * NBUF)],
        ],
    )
    def k(x_hbm, table_hbm, out_hbm, idx_v, *bufs_and_sems):
        rows = bufs_and_sems[0:NBUF]
        gsem = bufs_and_sems[NBUF:2 * NBUF]
        osem = bufs_and_sems[2 * NBUF:3 * NBUF]

        wid = lax.axis_index("s") * NC + lax.axis_index("c")
        base = wid * n_per_w
        pltpu.sync_copy(x_hbm.at[wid], idx_v)

        for g in range(PF):
            pltpu.async_copy(table_hbm.at[idx_v.at[g]], rows[g], gsem[g])

        def step(i, carry):
            for b0 in range(NBUF):
                gg = i * NBUF + b0
                b = b0
                bp = (b0 + PF) % NBUF

                pltpu.make_async_copy(table_hbm.at[pl.ds(0, G)],
                                      rows[b], gsem[b]).wait()

                @pl.when(gg + PF < n_groups)
                def _():
                    @pl.when(gg + PF >= NBUF)
                    def _():
                        pltpu.make_async_copy(
                            rows[bp],
                            out_hbm.at[pl.ds(base + (gg + PF - NBUF) * G, G)],
                            osem[bp]).wait()

                    pltpu.async_copy(table_hbm.at[idx_v.at[gg + PF]],
                                     rows[bp], gsem[bp])

                def row(r, c):
                    for j in range(D_MODEL // 16):
                        sl = pl.ds(j * 16, 16)
                        rows[b][r, sl] = rows[b][r, sl] * SCALE
                    return c

                lax.fori_loop(0, G, row, 0)

                pltpu.async_copy(rows[b],
                                 out_hbm.at[pl.ds(base + gg * G, G)],
                                 osem[b])
            return carry

        lax.fori_loop(0, n_groups // NBUF, step, 0)

        for b in range(NBUF):
            gg = n_groups - NBUF + b
            pltpu.make_async_copy(rows[b],
                                  out_hbm.at[pl.ds(base + gg * G, G)],
                                  osem[b]).wait()

    return k


def kernel(x, table):
    s0, s1 = x.shape
    B = s0 * s1
    xi = x.reshape(NW, B // (NW * G), G).astype(jnp.int32)
    out = _emb_kernel(B)(xi, table)
    return out.reshape(s0, s1, D_MODEL)


# speedup vs baseline: 1.5101x; 1.0004x over previous
import functools

import jax
import jax.numpy as jnp
from jax import lax
from jax.experimental import pallas as pl
from jax.experimental.pallas import tpu as pltpu
from jax.experimental.pallas import tpu_sc as plsc

D_MODEL = 128
SCALE = float(D_MODEL) ** 0.5

NC = 2
NS = 16
NW = NC * NS
G = 128
NBUF = 5
PF = 2


@functools.lru_cache(maxsize=None)
def _emb_kernel(B: int):
    n_per_w = B // NW
    n_groups = n_per_w // G
    assert n_groups % NBUF == 0

    mesh = plsc.VectorSubcoreMesh(core_axis_name="c", subcore_axis_name="s")

    @functools.partial(
        pl.kernel,
        mesh=mesh,
        out_type=jax.ShapeDtypeStruct((B, D_MODEL), jnp.float32),
        scratch_types=[
            pltpu.VMEM((n_groups, G), jnp.int32),
            *[pltpu.VMEM((G, D_MODEL), jnp.float32) for _ in range(NBUF)],
            *[pltpu.SemaphoreType.DMA for _ in range(2 * NBUF)],
        ],
    )
    def k(x_hbm, table_hbm, out_hbm, idx_v, *bufs_and_sems):
        rows = bufs_and_sems[0:NBUF]
        gsem = bufs_and_sems[NBUF:2 * NBUF]
        osem = bufs_and_sems[2 * NBUF:3 * NBUF]

        wid = lax.axis_index("s") * NC + lax.axis_index("c")
        base = wid * n_per_w
        pltpu.sync_copy(x_hbm.at[wid], idx_v)

        for g in range(PF):
            pltpu.async_copy(table_hbm.at[idx_v.at[g]], rows[g], gsem[g])

        def step(i, carry):
            for b0 in range(NBUF):
                gg = i * NBUF + b0
                b = b0
                bp = (b0 + PF) % NBUF

                pltpu.make_async_copy(table_hbm.at[pl.ds(0, G)],
                                      rows[b], gsem[b]).wait()

                @pl.when(gg + PF < n_groups)
                def _():
                    @pl.when(gg + PF >= NBUF)
                    def _():
                        pltpu.make_async_copy(
                            rows[bp],
                            out_hbm.at[pl.ds(base + (gg + PF - NBUF) * G, G)],
                            osem[bp]).wait()

                    pltpu.async_copy(table_hbm.at[idx_v.at[gg + PF]],
                                     rows[bp], gsem[bp])

                def row(r, c):
                    for j in range(D_MODEL // 16):
                        sl = pl.ds(j * 16, 16)
                        rows[b][r, sl] = rows[b][r, sl] * SCALE
                    return c

                lax.fori_loop(0, G, row, 0, unroll=4)

                pltpu.async_copy(rows[b],
                                 out_hbm.at[pl.ds(base + gg * G, G)],
                                 osem[b])
            return carry

        lax.fori_loop(0, n_groups // NBUF, step, 0)

        for b in range(NBUF):
            gg = n_groups - NBUF + b
            pltpu.make_async_copy(rows[b],
                                  out_hbm.at[pl.ds(base + gg * G, G)],
                                  osem[b]).wait()

    return k


def kernel(x, table):
    s0, s1 = x.shape
    B = s0 * s1
    xi = x.reshape(NW, B // (NW * G), G).astype(jnp.int32)
    out = _emb_kernel(B)(xi, table)
    return out.reshape(s0, s1, D_MODEL)


# speedup vs baseline: 1.5195x; 1.0062x over previous
import functools

import jax
import jax.numpy as jnp
from jax import lax
from jax.experimental import pallas as pl
from jax.experimental.pallas import tpu as pltpu
from jax.experimental.pallas import tpu_sc as plsc

D_MODEL = 128
SCALE = float(D_MODEL) ** 0.5

NC = 2
NS = 16
NW = NC * NS
G = 128
NBUF = 5
PF = 3


@functools.lru_cache(maxsize=None)
def _emb_kernel(B: int):
    n_per_w = B // NW
    n_groups = n_per_w // G
    assert n_groups % NBUF == 0

    mesh = plsc.VectorSubcoreMesh(core_axis_name="c", subcore_axis_name="s")

    @functools.partial(
        pl.kernel,
        mesh=mesh,
        out_type=jax.ShapeDtypeStruct((B, D_MODEL), jnp.float32),
        scratch_types=[
            pltpu.VMEM((n_groups, G), jnp.int32),
            *[pltpu.VMEM((G, D_MODEL), jnp.float32) for _ in range(NBUF)],
            *[pltpu.SemaphoreType.DMA for _ in range(2 * NBUF)],
        ],
    )
    def k(x_hbm, table_hbm, out_hbm, idx_v, *bufs_and_sems):
        rows = bufs_and_sems[0:NBUF]
        gsem = bufs_and_sems[NBUF:2 * NBUF]
        osem = bufs_and_sems[2 * NBUF:3 * NBUF]

        wid = lax.axis_index("s") * NC + lax.axis_index("c")
        base = wid * n_per_w
        pltpu.sync_copy(x_hbm.at[wid], idx_v)

        for g in range(PF):
            pltpu.async_copy(table_hbm.at[idx_v.at[g]], rows[g], gsem[g])

        def step(i, carry):
            for b0 in range(NBUF):
                gg = i * NBUF + b0
                b = b0
                bp = (b0 + PF) % NBUF

                pltpu.make_async_copy(table_hbm.at[pl.ds(0, G)],
                                      rows[b], gsem[b]).wait()

                @pl.when(gg + PF < n_groups)
                def _():
                    @pl.when(gg + PF >= NBUF)
                    def _():
                        pltpu.make_async_copy(
                            rows[bp],
                            out_hbm.at[pl.ds(base + (gg + PF - NBUF) * G, G)],
                            osem[bp]).wait()

                    pltpu.async_copy(table_hbm.at[idx_v.at[gg + PF]],
                                     rows[bp], gsem[bp])

                def row(r, c):
                    for j in range(D_MODEL // 16):
                        sl = pl.ds(j * 16, 16)
                        rows[b][r, sl] = rows[b][r, sl] * SCALE
                    return c

                lax.fori_loop(0, G, row, 0)

                pltpu.async_copy(rows[b],
                                 out_hbm.at[pl.ds(base + gg * G, G)],
                                 osem[b])
            return carry

        lax.fori_loop(0, n_groups // NBUF, step, 0)

        for b in range(NBUF):
            gg = n_groups - NBUF + b
            pltpu.make_async_copy(rows[b],
                                  out_hbm.at[pl.ds(base + gg * G, G)],
                                  osem[b]).wait()

    return k


def kernel(x, table):
    s0, s1 = x.shape
    B = s0 * s1
    xi = x.reshape(NW, B // (NW * G), G).astype(jnp.int32)
    out = _emb_kernel(B)(xi, table)
    return out.reshape(s0, s1, D_MODEL)
